# Initial kernel scaffold; baseline (speedup 1.0000x reference)
import jax
import jax.numpy as jnp
from jax.experimental import pallas as pl


def kernel(feat, nodes, globals_, ind, num, Wq, bq, Wk, bk, conv_w, conv_b, W1, b1, ln_gamma, ln_beta):
    raise NotImplementedError("write your pallas kernel here")


# speedup vs baseline: 18.3414x; 18.3414x over previous
import functools
import math

import jax
import jax.numpy as jnp
from jax import lax
from jax.experimental import pallas as pl
from jax.experimental.pallas import tpu as pltpu
from jax.experimental.pallas import tpu_sc as plsc

E = 400000
N = 100000
G = 64
K = 16
D = 128
H = 8
HID = 128
LN_EPS = 1e-3

BE = 3200
NBE = E // BE
BN = 4000
NBN = N // BN
CHUNK = 128
CPB = BE // CHUNK
NUM_CHUNKS = E // CHUNK
NQ = 4


def _qm_body(feat_ref, wq_ref, cw1_ref, bq_ref, cb_ref, out_ref):
    wqp = jnp.dot(wq_ref[...], cw1_ref[...], preferred_element_type=jnp.float32)
    bias = jnp.dot(bq_ref[...], cw1_ref[...], preferred_element_type=jnp.float32) + cb_ref[...]
    q = jnp.dot(feat_ref[...], wqp, preferred_element_type=jnp.float32) + bias
    out_ref[...] = jnp.transpose(q)[None, :, :]


def _km_body(nodes_ref, wk_ref, cw1_ref, bk_ref, cb_ref, glob_ref, w1c_ref, b1_ref,
             km_ref, gc_ref):
    i = pl.program_id(0)
    wkp = jnp.dot(wk_ref[...], cw1_ref[...], preferred_element_type=jnp.float32)
    bias = jnp.dot(bk_ref[...], cw1_ref[...], preferred_element_type=jnp.float32) + cb_ref[...]
    kmb = jnp.dot(nodes_ref[...], wkp, preferred_element_type=jnp.float32) + bias
    km_ref[...] = jnp.transpose(kmb)[None, :, :]

    @pl.when(i == 0)
    def _():
        gc_ref[...] = (
            jnp.dot(glob_ref[...], w1c_ref[...], preferred_element_type=jnp.float32)
            + b1_ref[...]
        )


def _sc_attn_body(ind_flat, qmT, kmT, out, tab_v, idx_v, qh_v, oh_v):
    wid = lax.axis_index("c") * 16 + lax.axis_index("s")
    h = wid % H
    qtr = wid // H

    for nb in range(NBN):
        pltpu.sync_copy(
            kmT.at[pl.ds((nb * H + h) * BN, BN)], tab_v.at[pl.ds(nb * BN, BN)]
        )

    iota16 = lax.iota(jnp.int32, 16)
    riota = iota16 * K
    inv_sqrt_h = 1.0 / math.sqrt(float(H))

    def chunk_body(i, carry):
        c = qtr + i * NQ
        nb = c // CPB
        off = (c % CPB) * CHUNK
        qbase = (nb * H + h) * BE + off
        pltpu.sync_copy(ind_flat.at[pl.ds(c * (CHUNK * K), CHUNK * K)], idx_v)
        pltpu.sync_copy(qmT.at[pl.ds(qbase, CHUNK)], qh_v)
        for g in range(CHUNK // 16):
            q = qh_v[pl.ds(g * 16, 16)] * inv_sqrt_h
            den = jnp.zeros((16,), jnp.float32)
            num = jnp.zeros((16,), jnp.float32)
            for k in range(K):
                iv = plsc.load_gather(idx_v, [riota + (g * 256 + k)])
                gk = plsc.load_gather(tab_v, [iv])
                t = jnp.exp(q * gk)
                den = den + t
                num = num + t * gk
            oh_v[pl.ds(g * 16, 16)] = num / den
        pltpu.sync_copy(oh_v, out.at[pl.ds(qbase, CHUNK)])
        return carry

    n_c = (NUM_CHUNKS - qtr + NQ - 1) // NQ
    lax.fori_loop(0, n_c, chunk_body, 0)


def _fin_body(feat_ref, mh_ref, w1a_ref, w1b_ref, gc_ref, starts_ref, ends_ref,
              gamma_ref, beta_ref, out_ref):
    i = pl.program_id(0)
    mh = jnp.transpose(mh_ref[0])
    acc = jnp.dot(feat_ref[...], w1a_ref[...], preferred_element_type=jnp.float32)
    acc = acc + jnp.dot(mh, w1b_ref[...], preferred_element_type=jnp.float32)
    rows = lax.broadcasted_iota(jnp.int32, (BE, G), 0) + i * BE
    onehot = ((rows >= starts_ref[...]) & (rows < ends_ref[...])).astype(jnp.float32)
    acc = acc + jnp.dot(onehot, gc_ref[...], preferred_element_type=jnp.float32)
    hh = jnp.maximum(acc, 0.0)
    mu = jnp.mean(hh, axis=1, keepdims=True)
    dd = hh - mu
    var = jnp.mean(dd * dd, axis=1, keepdims=True)
    out_ref[...] = dd * lax.rsqrt(var + LN_EPS) * gamma_ref[...] + beta_ref[...]


def kernel(feat, nodes, globals_, ind, num, Wq, bq, Wk, bk, conv_w, conv_b,
           W1, b1, ln_gamma, ln_beta):
    cw1 = conv_w[1]
    bq2 = bq.reshape(1, -1)
    bk2 = bk.reshape(1, -1)
    cb2 = conv_b.reshape(1, -1)
    w1a = W1[:D]
    w1b = W1[D:D + H]
    w1c = W1[D + H:]
    b12 = b1.reshape(1, -1)
    csum = jnp.cumsum(num.astype(jnp.int32))
    starts = (csum - num.astype(jnp.int32)).reshape(1, G)
    ends = csum.reshape(1, G)
    gamma2 = ln_gamma.reshape(1, -1)
    beta2 = ln_beta.reshape(1, -1)

    qmT = pl.pallas_call(
        _qm_body,
        grid=(NBE,),
        in_specs=[
            pl.BlockSpec((BE, D), lambda i: (i, 0)),
            pl.BlockSpec((D, H), lambda i: (0, 0)),
            pl.BlockSpec((H, H), lambda i: (0, 0)),
            pl.BlockSpec((1, H), lambda i: (0, 0)),
            pl.BlockSpec((1, H), lambda i: (0, 0)),
        ],
        out_specs=pl.BlockSpec((1, H, BE), lambda i: (i, 0, 0)),
        out_shape=jax.ShapeDtypeStruct((NBE, H, BE), jnp.float32),
    )(feat, Wq, cw1, bq2, cb2)

    kmT, gc = pl.pallas_call(
        _km_body,
        grid=(NBN,),
        in_specs=[
            pl.BlockSpec((BN, D), lambda i: (i, 0)),
            pl.BlockSpec((D, H), lambda i: (0, 0)),
            pl.BlockSpec((H, H), lambda i: (0, 0)),
            pl.BlockSpec((1, H), lambda i: (0, 0)),
            pl.BlockSpec((1, H), lambda i: (0, 0)),
            pl.BlockSpec((G, D), lambda i: (0, 0)),
            pl.BlockSpec((D, HID), lambda i: (0, 0)),
            pl.BlockSpec((1, HID), lambda i: (0, 0)),
        ],
        out_specs=[
            pl.BlockSpec((1, H, BN), lambda i: (i, 0, 0)),
            pl.BlockSpec((G, HID), lambda i: (0, 0)),
        ],
        out_shape=[
            jax.ShapeDtypeStruct((NBN, H, BN), jnp.float32),
            jax.ShapeDtypeStruct((G, HID), jnp.float32),
        ],
    )(nodes, Wk, cw1, bk2, cb2, globals_, w1c, b12)

    ind_flat = ind.astype(jnp.int32).reshape(-1)
    qmT_flat = qmT.reshape(-1)
    kmT_flat = kmT.reshape(-1)
    sc_attn = functools.partial(
        pl.kernel,
        mesh=plsc.VectorSubcoreMesh(core_axis_name="c", subcore_axis_name="s"),
        compiler_params=pltpu.CompilerParams(needs_layout_passes=False),
        out_type=jax.ShapeDtypeStruct((E * H,), jnp.float32),
        scratch_types=[
            pltpu.VMEM((N,), jnp.float32),
            pltpu.VMEM((CHUNK * K,), jnp.int32),
            pltpu.VMEM((CHUNK,), jnp.float32),
            pltpu.VMEM((CHUNK,), jnp.float32),
        ],
    )(_sc_attn_body)
    mhT = sc_attn(ind_flat, qmT_flat, kmT_flat).reshape(NBE, H, BE)

    out = pl.pallas_call(
        _fin_body,
        grid=(NBE,),
        in_specs=[
            pl.BlockSpec((BE, D), lambda i: (i, 0)),
            pl.BlockSpec((1, H, BE), lambda i: (i, 0, 0)),
            pl.BlockSpec((D, HID), lambda i: (0, 0)),
            pl.BlockSpec((H, HID), lambda i: (0, 0)),
            pl.BlockSpec((G, HID), lambda i: (0, 0)),
            pl.BlockSpec((1, G), lambda i: (0, 0)),
            pl.BlockSpec((1, G), lambda i: (0, 0)),
            pl.BlockSpec((1, HID), lambda i: (0, 0)),
            pl.BlockSpec((1, HID), lambda i: (0, 0)),
        ],
        out_specs=pl.BlockSpec((BE, HID), lambda i: (i, 0)),
        out_shape=jax.ShapeDtypeStruct((E, HID), jnp.float32),
    )(feat, mhT, w1a, w1b, gc, starts, ends, gamma2, beta2)
    return out


# speedup vs baseline: 34.5675x; 1.8847x over previous
import functools
import math

import jax
import jax.numpy as jnp
from jax import lax
from jax.experimental import pallas as pl
from jax.experimental.pallas import tpu as pltpu
from jax.experimental.pallas import tpu_sc as plsc

E = 400000
N = 100000
G = 64
K = 16
D = 128
H = 8
HID = 128
LN_EPS = 1e-3

BE = 3200
NBE = E // BE
BN = 4000
NBN = N // BN
CHUNK = 640
CPB = BE // CHUNK
NUM_CHUNKS = E // CHUNK
NQ = 4
GROUPS = CHUNK // 16
MAX_NC = (NUM_CHUNKS + NQ - 1) // NQ


def _qm_body(feat_ref, wq_ref, cw1_ref, bq_ref, cb_ref, out_ref):
    wqp = jnp.dot(wq_ref[...], cw1_ref[...], preferred_element_type=jnp.float32)
    bias = jnp.dot(bq_ref[...], cw1_ref[...], preferred_element_type=jnp.float32) + cb_ref[...]
    q = jnp.dot(feat_ref[...], wqp, preferred_element_type=jnp.float32) + bias
    out_ref[...] = jnp.transpose(q)[None, :, :]


def _km_body(nodes_ref, wk_ref, cw1_ref, bk_ref, cb_ref, glob_ref, w1c_ref, b1_ref,
             km_ref, gc_ref):
    i = pl.program_id(0)
    wkp = jnp.dot(wk_ref[...], cw1_ref[...], preferred_element_type=jnp.float32)
    bias = jnp.dot(bk_ref[...], cw1_ref[...], preferred_element_type=jnp.float32) + cb_ref[...]
    kmb = jnp.dot(nodes_ref[...], wkp, preferred_element_type=jnp.float32) + bias
    km_ref[...] = jnp.transpose(kmb)[None, :, :]

    @pl.when(i == 0)
    def _():
        gc_ref[...] = (
            jnp.dot(glob_ref[...], w1c_ref[...], preferred_element_type=jnp.float32)
            + b1_ref[...]
        )


def _sc_attn_body(ind_flat, qmT, kmT, out, tab_v, idx_v0, idx_v1, qh_v0, qh_v1,
                  oh_v0, oh_v1, sem_i0, sem_i1, sem_q0, sem_q1, sem_o0, sem_o1):
    idx_b = [idx_v0, idx_v1]
    qh_b = [qh_v0, qh_v1]
    oh_b = [oh_v0, oh_v1]
    sem_i = [sem_i0, sem_i1]
    sem_q = [sem_q0, sem_q1]
    sem_o = [sem_o0, sem_o1]
    wid = lax.axis_index("c") * 16 + lax.axis_index("s")
    h = wid % H
    qtr = wid // H

    for nb in range(NBN):
        pltpu.sync_copy(
            kmT.at[pl.ds((nb * H + h) * BN, BN)], tab_v.at[pl.ds(nb * BN, BN)]
        )

    iota16 = lax.iota(jnp.int32, 16)
    riota = iota16 * K
    qscale = 1.0 / math.sqrt(float(H))
    n_c = (NUM_CHUNKS - qtr + NQ - 1) // NQ

    def srcs(j):
        c = qtr + jnp.minimum(j, n_c - 1) * NQ
        nb = c // CPB
        off = (c % CPB) * CHUNK
        qbase = (nb * H + h) * BE + off
        return c, qbase

    def issue(j, b):
        c, qbase = srcs(j)
        pltpu.async_copy(
            ind_flat.at[pl.ds(c * (CHUNK * K), CHUNK * K)], idx_b[b], sem_i[b]
        )
        pltpu.async_copy(qmT.at[pl.ds(qbase, CHUNK)], qh_b[b], sem_q[b])

    def drain(j, b):
        c, qbase = srcs(j)
        pltpu.make_async_copy(
            ind_flat.at[pl.ds(c * (CHUNK * K), CHUNK * K)], idx_b[b], sem_i[b]
        ).wait()
        pltpu.make_async_copy(
            qmT.at[pl.ds(qbase, CHUNK)], qh_b[b], sem_q[b]
        ).wait()

    def compute(j, b):
        _, qbase = srcs(j)

        def group_body(g, carry2):
            q2 = qh_b[b][pl.ds(g * 16, 16)] * qscale
            den = jnp.zeros((16,), jnp.float32)
            num = jnp.zeros((16,), jnp.float32)
            for k in range(K):
                iv = plsc.load_gather(idx_b[b], [riota + (g * 256 + k)])
                gk = plsc.load_gather(tab_v, [iv])
                t = jnp.exp(q2 * gk)
                den = den + t
                num = num + t * gk
            oh_b[b][pl.ds(g * 16, 16)] = num / den
            return carry2

        lax.fori_loop(0, GROUPS, group_body, 0)
        pltpu.async_copy(oh_b[b], out.at[pl.ds(qbase, CHUNK)], sem_o[b])

    def drain_out(j, b):
        _, qbase = srcs(j)
        pltpu.make_async_copy(
            oh_b[b], out.at[pl.ds(qbase, CHUNK)], sem_o[b]
        ).wait()

    issue(0, 0)

    def outer(i, carry):
        for b in range(2):
            j = i * 2 + b
            drain(j, b)
            issue(j + 1, 1 - b)

            @pl.when(j >= 2)
            def _():
                drain_out(j - 2, b)

            compute(j, b)
        return carry

    lax.fori_loop(0, MAX_NC // 2, outer, 0)
    j_last = MAX_NC - 1
    b_last = j_last % 2
    drain(j_last, b_last)
    drain_out(j_last - 2, b_last)
    compute(j_last, b_last)
    drain_out(j_last - 1, 1 - b_last)
    drain_out(j_last, b_last)


def _fin_body(feat_ref, mh_ref, w1a_ref, w1b_ref, gc_ref, starts_ref, ends_ref,
              gamma_ref, beta_ref, out_ref):
    i = pl.program_id(0)
    mh = jnp.transpose(mh_ref[0])
    acc = jnp.dot(feat_ref[...], w1a_ref[...], preferred_element_type=jnp.float32)
    acc = acc + jnp.dot(mh, w1b_ref[...], preferred_element_type=jnp.float32)
    rows = lax.broadcasted_iota(jnp.int32, (BE, G), 0) + i * BE
    onehot = ((rows >= starts_ref[...]) & (rows < ends_ref[...])).astype(jnp.float32)
    acc = acc + jnp.dot(onehot, gc_ref[...], preferred_element_type=jnp.float32)
    hh = jnp.maximum(acc, 0.0)
    mu = jnp.mean(hh, axis=1, keepdims=True)
    dd = hh - mu
    var = jnp.mean(dd * dd, axis=1, keepdims=True)
    out_ref[...] = dd * lax.rsqrt(var + LN_EPS) * gamma_ref[...] + beta_ref[...]


def kernel(feat, nodes, globals_, ind, num, Wq, bq, Wk, bk, conv_w, conv_b,
           W1, b1, ln_gamma, ln_beta):
    cw1 = conv_w[1]
    bq2 = bq.reshape(1, -1)
    bk2 = bk.reshape(1, -1)
    cb2 = conv_b.reshape(1, -1)
    w1a = W1[:D]
    w1b = W1[D:D + H]
    w1c = W1[D + H:]
    b12 = b1.reshape(1, -1)
    csum = jnp.cumsum(num.astype(jnp.int32))
    starts = (csum - num.astype(jnp.int32)).reshape(1, G)
    ends = csum.reshape(1, G)
    gamma2 = ln_gamma.reshape(1, -1)
    beta2 = ln_beta.reshape(1, -1)

    qmT = pl.pallas_call(
        _qm_body,
        grid=(NBE,),
        in_specs=[
            pl.BlockSpec((BE, D), lambda i: (i, 0)),
            pl.BlockSpec((D, H), lambda i: (0, 0)),
            pl.BlockSpec((H, H), lambda i: (0, 0)),
            pl.BlockSpec((1, H), lambda i: (0, 0)),
            pl.BlockSpec((1, H), lambda i: (0, 0)),
        ],
        out_specs=pl.BlockSpec((1, H, BE), lambda i: (i, 0, 0)),
        out_shape=jax.ShapeDtypeStruct((NBE, H, BE), jnp.float32),
    )(feat, Wq, cw1, bq2, cb2)

    kmT, gc = pl.pallas_call(
        _km_body,
        grid=(NBN,),
        in_specs=[
            pl.BlockSpec((BN, D), lambda i: (i, 0)),
            pl.BlockSpec((D, H), lambda i: (0, 0)),
            pl.BlockSpec((H, H), lambda i: (0, 0)),
            pl.BlockSpec((1, H), lambda i: (0, 0)),
            pl.BlockSpec((1, H), lambda i: (0, 0)),
            pl.BlockSpec((G, D), lambda i: (0, 0)),
            pl.BlockSpec((D, HID), lambda i: (0, 0)),
            pl.BlockSpec((1, HID), lambda i: (0, 0)),
        ],
        out_specs=[
            pl.BlockSpec((1, H, BN), lambda i: (i, 0, 0)),
            pl.BlockSpec((G, HID), lambda i: (0, 0)),
        ],
        out_shape=[
            jax.ShapeDtypeStruct((NBN, H, BN), jnp.float32),
            jax.ShapeDtypeStruct((G, HID), jnp.float32),
        ],
    )(nodes, Wk, cw1, bk2, cb2, globals_, w1c, b12)

    ind_flat = ind.astype(jnp.int32).reshape(-1)
    qmT_flat = qmT.reshape(-1)
    kmT_flat = kmT.reshape(-1)
    sc_attn = functools.partial(
        pl.kernel,
        mesh=plsc.VectorSubcoreMesh(core_axis_name="c", subcore_axis_name="s"),
        compiler_params=pltpu.CompilerParams(needs_layout_passes=False),
        out_type=jax.ShapeDtypeStruct((E * H,), jnp.float32),
        scratch_types=(
            [pltpu.VMEM((N,), jnp.float32)]
            + [pltpu.VMEM((CHUNK * K,), jnp.int32)] * 2
            + [pltpu.VMEM((CHUNK,), jnp.float32)] * 4
            + [pltpu.SemaphoreType.DMA] * 6
        ),
    )(_sc_attn_body)
    mhT = sc_attn(ind_flat, qmT_flat, kmT_flat).reshape(NBE, H, BE)

    out = pl.pallas_call(
        _fin_body,
        grid=(NBE,),
        in_specs=[
            pl.BlockSpec((BE, D), lambda i: (i, 0)),
            pl.BlockSpec((1, H, BE), lambda i: (i, 0, 0)),
            pl.BlockSpec((D, HID), lambda i: (0, 0)),
            pl.BlockSpec((H, HID), lambda i: (0, 0)),
            pl.BlockSpec((G, HID), lambda i: (0, 0)),
            pl.BlockSpec((1, G), lambda i: (0, 0)),
            pl.BlockSpec((1, G), lambda i: (0, 0)),
            pl.BlockSpec((1, HID), lambda i: (0, 0)),
            pl.BlockSpec((1, HID), lambda i: (0, 0)),
        ],
        out_specs=pl.BlockSpec((BE, HID), lambda i: (i, 0)),
        out_shape=jax.ShapeDtypeStruct((E, HID), jnp.float32),
    )(feat, mhT, w1a, w1b, gc, starts, ends, gamma2, beta2)
    return out


# speedup vs baseline: 42.8442x; 1.2394x over previous
import functools
import math

import jax
import jax.numpy as jnp
from jax import lax
from jax.experimental import pallas as pl
from jax.experimental.pallas import tpu as pltpu
from jax.experimental.pallas import tpu_sc as plsc

E = 400000
N = 100000
G = 64
K = 16
D = 128
H = 8
HID = 128
LN_EPS = 1e-3

BE = 3200
NBE = E // BE
BN = 4000
NBN = N // BN
CHUNK = 640
CPB = BE // CHUNK
NUM_CHUNKS = E // CHUNK
NQ = 4
GROUPS = CHUNK // 16
MAX_NC = (NUM_CHUNKS + NQ - 1) // NQ


def _qm_body(feat_ref, wq_ref, cw1_ref, bq_ref, cb_ref, out_ref):
    wqp = jnp.dot(wq_ref[...], cw1_ref[...], preferred_element_type=jnp.float32)
    bias = jnp.dot(bq_ref[...], cw1_ref[...], preferred_element_type=jnp.float32) + cb_ref[...]
    q = jnp.dot(feat_ref[...], wqp, preferred_element_type=jnp.float32) + bias
    out_ref[...] = jnp.transpose(q)[None, :, :]


def _km_body(nodes_ref, wk_ref, cw1_ref, bk_ref, cb_ref, glob_ref, w1c_ref, b1_ref,
             km_ref, gc_ref):
    i = pl.program_id(0)
    wkp = jnp.dot(wk_ref[...], cw1_ref[...], preferred_element_type=jnp.float32)
    bias = jnp.dot(bk_ref[...], cw1_ref[...], preferred_element_type=jnp.float32) + cb_ref[...]
    kmb = jnp.dot(nodes_ref[...], wkp, preferred_element_type=jnp.float32) + bias
    km_ref[...] = jnp.transpose(kmb)[None, :, :]

    @pl.when(i == 0)
    def _():
        gc_ref[...] = (
            jnp.dot(glob_ref[...], w1c_ref[...], preferred_element_type=jnp.float32)
            + b1_ref[...]
        )


def _sc_attn_body(ind_flat, qmT, kmT, out, tab_v, idx_v0, idx_v1, qh_v0, qh_v1,
                  oh_v0, oh_v1, sem_i0, sem_i1, sem_q0, sem_q1, sem_o0, sem_o1):
    idx_b = [idx_v0, idx_v1]
    qh_b = [qh_v0, qh_v1]
    oh_b = [oh_v0, oh_v1]
    sem_i = [sem_i0, sem_i1]
    sem_q = [sem_q0, sem_q1]
    sem_o = [sem_o0, sem_o1]
    wid = lax.axis_index("c") * 16 + lax.axis_index("s")
    h = wid % H
    qtr = wid // H

    for nb in range(NBN):
        pltpu.sync_copy(
            kmT.at[pl.ds((nb * H + h) * BN, BN)], tab_v.at[pl.ds(nb * BN, BN)]
        )

    qscale = 1.0 / math.sqrt(float(H))
    n_c = (NUM_CHUNKS - qtr + NQ - 1) // NQ

    def srcs(j):
        c = qtr + jnp.minimum(j, n_c - 1) * NQ
        nb = c // CPB
        off = (c % CPB) * CHUNK
        qbase = (nb * H + h) * BE + off
        return c, qbase

    def issue(j, b):
        c, qbase = srcs(j)
        pltpu.async_copy(
            ind_flat.at[pl.ds(c * (CHUNK * K), CHUNK * K)], idx_b[b], sem_i[b]
        )
        pltpu.async_copy(qmT.at[pl.ds(qbase, CHUNK)], qh_b[b], sem_q[b])

    def drain(j, b):
        c, qbase = srcs(j)
        pltpu.make_async_copy(
            ind_flat.at[pl.ds(c * (CHUNK * K), CHUNK * K)], idx_b[b], sem_i[b]
        ).wait()
        pltpu.make_async_copy(
            qmT.at[pl.ds(qbase, CHUNK)], qh_b[b], sem_q[b]
        ).wait()

    def compute(j, b):
        _, qbase = srcs(j)

        def group_body(g, carry2):
            q2 = qh_b[b][pl.ds(g * 16, 16)] * qscale
            den = jnp.zeros((16,), jnp.float32)
            num = jnp.zeros((16,), jnp.float32)
            for k in range(K):
                iv = idx_b[b][pl.ds(k * CHUNK + g * 16, 16)]
                gk = plsc.load_gather(tab_v, [iv])
                t = jnp.exp(q2 * gk)
                den = den + t
                num = num + t * gk
            oh_b[b][pl.ds(g * 16, 16)] = num / den
            return carry2

        lax.fori_loop(0, GROUPS, group_body, 0)
        pltpu.async_copy(oh_b[b], out.at[pl.ds(qbase, CHUNK)], sem_o[b])

    def drain_out(j, b):
        _, qbase = srcs(j)
        pltpu.make_async_copy(
            oh_b[b], out.at[pl.ds(qbase, CHUNK)], sem_o[b]
        ).wait()

    issue(0, 0)

    def outer(i, carry):
        for b in range(2):
            j = i * 2 + b
            drain(j, b)
            issue(j + 1, 1 - b)

            @pl.when(j >= 2)
            def _():
                drain_out(j - 2, b)

            compute(j, b)
        return carry

    lax.fori_loop(0, MAX_NC // 2, outer, 0)
    j_last = MAX_NC - 1
    b_last = j_last % 2
    drain(j_last, b_last)
    drain_out(j_last - 2, b_last)
    compute(j_last, b_last)
    drain_out(j_last - 1, 1 - b_last)
    drain_out(j_last, b_last)


def _fin_body(feat_ref, mh_ref, w1a_ref, w1b_ref, gc_ref, starts_ref, ends_ref,
              gamma_ref, beta_ref, out_ref):
    i = pl.program_id(0)
    mh = jnp.transpose(mh_ref[0])
    acc = jnp.dot(feat_ref[...], w1a_ref[...], preferred_element_type=jnp.float32)
    acc = acc + jnp.dot(mh, w1b_ref[...], preferred_element_type=jnp.float32)
    rows = lax.broadcasted_iota(jnp.int32, (BE, G), 0) + i * BE
    onehot = ((rows >= starts_ref[...]) & (rows < ends_ref[...])).astype(jnp.float32)
    acc = acc + jnp.dot(onehot, gc_ref[...], preferred_element_type=jnp.float32)
    hh = jnp.maximum(acc, 0.0)
    mu = jnp.mean(hh, axis=1, keepdims=True)
    dd = hh - mu
    var = jnp.mean(dd * dd, axis=1, keepdims=True)
    out_ref[...] = dd * lax.rsqrt(var + LN_EPS) * gamma_ref[...] + beta_ref[...]


def kernel(feat, nodes, globals_, ind, num, Wq, bq, Wk, bk, conv_w, conv_b,
           W1, b1, ln_gamma, ln_beta):
    cw1 = conv_w[1]
    bq2 = bq.reshape(1, -1)
    bk2 = bk.reshape(1, -1)
    cb2 = conv_b.reshape(1, -1)
    w1a = W1[:D]
    w1b = W1[D:D + H]
    w1c = W1[D + H:]
    b12 = b1.reshape(1, -1)
    csum = jnp.cumsum(num.astype(jnp.int32))
    starts = (csum - num.astype(jnp.int32)).reshape(1, G)
    ends = csum.reshape(1, G)
    gamma2 = ln_gamma.reshape(1, -1)
    beta2 = ln_beta.reshape(1, -1)

    qmT = pl.pallas_call(
        _qm_body,
        grid=(NBE,),
        in_specs=[
            pl.BlockSpec((BE, D), lambda i: (i, 0)),
            pl.BlockSpec((D, H), lambda i: (0, 0)),
            pl.BlockSpec((H, H), lambda i: (0, 0)),
            pl.BlockSpec((1, H), lambda i: (0, 0)),
            pl.BlockSpec((1, H), lambda i: (0, 0)),
        ],
        out_specs=pl.BlockSpec((1, H, BE), lambda i: (i, 0, 0)),
        out_shape=jax.ShapeDtypeStruct((NBE, H, BE), jnp.float32),
    )(feat, Wq, cw1, bq2, cb2)

    kmT, gc = pl.pallas_call(
        _km_body,
        grid=(NBN,),
        in_specs=[
            pl.BlockSpec((BN, D), lambda i: (i, 0)),
            pl.BlockSpec((D, H), lambda i: (0, 0)),
            pl.BlockSpec((H, H), lambda i: (0, 0)),
            pl.BlockSpec((1, H), lambda i: (0, 0)),
            pl.BlockSpec((1, H), lambda i: (0, 0)),
            pl.BlockSpec((G, D), lambda i: (0, 0)),
            pl.BlockSpec((D, HID), lambda i: (0, 0)),
            pl.BlockSpec((1, HID), lambda i: (0, 0)),
        ],
        out_specs=[
            pl.BlockSpec((1, H, BN), lambda i: (i, 0, 0)),
            pl.BlockSpec((G, HID), lambda i: (0, 0)),
        ],
        out_shape=[
            jax.ShapeDtypeStruct((NBN, H, BN), jnp.float32),
            jax.ShapeDtypeStruct((G, HID), jnp.float32),
        ],
    )(nodes, Wk, cw1, bk2, cb2, globals_, w1c, b12)

    ind_flat = (
        ind.astype(jnp.int32)
        .reshape(NUM_CHUNKS, CHUNK, K)
        .transpose(0, 2, 1)
        .reshape(-1)
    )
    qmT_flat = qmT.reshape(-1)
    kmT_flat = kmT.reshape(-1)
    sc_attn = functools.partial(
        pl.kernel,
        mesh=plsc.VectorSubcoreMesh(core_axis_name="c", subcore_axis_name="s"),
        compiler_params=pltpu.CompilerParams(needs_layout_passes=False),
        out_type=jax.ShapeDtypeStruct((E * H,), jnp.float32),
        scratch_types=(
            [pltpu.VMEM((N,), jnp.float32)]
            + [pltpu.VMEM((CHUNK * K,), jnp.int32)] * 2
            + [pltpu.VMEM((CHUNK,), jnp.float32)] * 4
            + [pltpu.SemaphoreType.DMA] * 6
        ),
    )(_sc_attn_body)
    mhT = sc_attn(ind_flat, qmT_flat, kmT_flat).reshape(NBE, H, BE)

    out = pl.pallas_call(
        _fin_body,
        grid=(NBE,),
        in_specs=[
            pl.BlockSpec((BE, D), lambda i: (i, 0)),
            pl.BlockSpec((1, H, BE), lambda i: (i, 0, 0)),
            pl.BlockSpec((D, HID), lambda i: (0, 0)),
            pl.BlockSpec((H, HID), lambda i: (0, 0)),
            pl.BlockSpec((G, HID), lambda i: (0, 0)),
            pl.BlockSpec((1, G), lambda i: (0, 0)),
            pl.BlockSpec((1, G), lambda i: (0, 0)),
            pl.BlockSpec((1, HID), lambda i: (0, 0)),
            pl.BlockSpec((1, HID), lambda i: (0, 0)),
        ],
        out_specs=pl.BlockSpec((BE, HID), lambda i: (i, 0)),
        out_shape=jax.ShapeDtypeStruct((E, HID), jnp.float32),
    )(feat, mhT, w1a, w1b, gc, starts, ends, gamma2, beta2)
    return out


# speedup vs baseline: 45.6581x; 1.0657x over previous
import functools
import math

import jax
import jax.numpy as jnp
from jax import lax
from jax.experimental import pallas as pl
from jax.experimental.pallas import tpu as pltpu
from jax.experimental.pallas import tpu_sc as plsc

E = 400000
N = 100000
G = 64
K = 16
D = 128
H = 8
HID = 128
LN_EPS = 1e-3

BE = 3200
NBE = E // BE
BN = 4000
NBN = N // BN
CHUNK = 640
CPB = BE // CHUNK
NUM_CHUNKS = E // CHUNK
NQ = 4
GROUPS = CHUNK // 16
SPLITS = 5
SPB = NBE // SPLITS
SPLIT_CHUNKS = NUM_CHUNKS // SPLITS
SPLIT_E = E // SPLITS
MAX_NC = (SPLIT_CHUNKS + NQ - 1) // NQ


def _qm_body(feat_ref, wq_ref, cw1_ref, bq_ref, cb_ref, out_ref):
    wqp = jnp.dot(wq_ref[...], cw1_ref[...], preferred_element_type=jnp.float32)
    bias = jnp.dot(bq_ref[...], cw1_ref[...], preferred_element_type=jnp.float32) + cb_ref[...]
    q = jnp.dot(feat_ref[...], wqp, preferred_element_type=jnp.float32) + bias
    out_ref[...] = jnp.transpose(q)[None, :, :]


def _km_body(nodes_ref, wk_ref, cw1_ref, bk_ref, cb_ref, glob_ref, w1c_ref, b1_ref,
             km_ref, gc_ref):
    i = pl.program_id(0)
    wkp = jnp.dot(wk_ref[...], cw1_ref[...], preferred_element_type=jnp.float32)
    bias = jnp.dot(bk_ref[...], cw1_ref[...], preferred_element_type=jnp.float32) + cb_ref[...]
    kmb = jnp.dot(nodes_ref[...], wkp, preferred_element_type=jnp.float32) + bias
    km_ref[...] = jnp.transpose(kmb)[None, :, :]

    @pl.when(i == 0)
    def _():
        gc_ref[...] = (
            jnp.dot(glob_ref[...], w1c_ref[...], preferred_element_type=jnp.float32)
            + b1_ref[...]
        )


def _make_sc_body(s):
    def body(ind_flat, qmT, kmT, out, tab_v, idx_v0, idx_v1, qh_v0, qh_v1,
             oh_v0, oh_v1, sem_i0, sem_i1, sem_q0, sem_q1, sem_o0, sem_o1):
        idx_b = [idx_v0, idx_v1]
        qh_b = [qh_v0, qh_v1]
        oh_b = [oh_v0, oh_v1]
        sem_i = [sem_i0, sem_i1]
        sem_q = [sem_q0, sem_q1]
        sem_o = [sem_o0, sem_o1]
        wid = lax.axis_index("c") * 16 + lax.axis_index("s")
        h = wid % H
        qtr = wid // H

        for nb in range(NBN):
            pltpu.sync_copy(
                kmT.at[pl.ds((nb * H + h) * BN, BN)], tab_v.at[pl.ds(nb * BN, BN)]
            )

        qscale = 1.0 / math.sqrt(float(H))
        n_c = (SPLIT_CHUNKS - qtr + NQ - 1) // NQ

        def srcs(j):
            lc = qtr + jnp.minimum(j, n_c - 1) * NQ
            c = s * SPLIT_CHUNKS + lc
            off = (lc % CPB) * CHUNK
            qbase = ((c // CPB) * H + h) * BE + off
            obase = ((lc // CPB) * H + h) * BE + off
            return c, qbase, obase

        def issue(j, b):
            c, qbase, _ = srcs(j)
            pltpu.async_copy(
                ind_flat.at[pl.ds(c * (CHUNK * K), CHUNK * K)], idx_b[b], sem_i[b]
            )
            pltpu.async_copy(qmT.at[pl.ds(qbase, CHUNK)], qh_b[b], sem_q[b])

        def drain(j, b):
            c, qbase, _ = srcs(j)
            pltpu.make_async_copy(
                ind_flat.at[pl.ds(c * (CHUNK * K), CHUNK * K)], idx_b[b], sem_i[b]
            ).wait()
            pltpu.make_async_copy(
                qmT.at[pl.ds(qbase, CHUNK)], qh_b[b], sem_q[b]
            ).wait()

        def compute(j, b):
            _, _, obase = srcs(j)

            def group_body(g, carry2):
                q2 = qh_b[b][pl.ds(g * 16, 16)] * qscale
                den = jnp.zeros((16,), jnp.float32)
                num = jnp.zeros((16,), jnp.float32)
                for k in range(K):
                    iv = idx_b[b][pl.ds(k * CHUNK + g * 16, 16)]
                    gk = plsc.load_gather(tab_v, [iv])
                    t = jnp.exp(q2 * gk)
                    den = den + t
                    num = num + t * gk
                oh_b[b][pl.ds(g * 16, 16)] = num / den
                return carry2

            lax.fori_loop(0, GROUPS, group_body, 0)
            pltpu.async_copy(oh_b[b], out.at[pl.ds(obase, CHUNK)], sem_o[b])

        def drain_out(j, b):
            _, _, obase = srcs(j)
            pltpu.make_async_copy(
                oh_b[b], out.at[pl.ds(obase, CHUNK)], sem_o[b]
            ).wait()

        issue(0, 0)

        def outer(i, carry):
            for b in range(2):
                j = i * 2 + b
                drain(j, b)
                issue(j + 1, 1 - b)

                @pl.when(j >= 2)
                def _():
                    drain_out(j - 2, b)

                compute(j, b)
            return carry

        lax.fori_loop(0, MAX_NC // 2, outer, 0)
        drain(MAX_NC, 0)
        drain_out(MAX_NC - 2, 0)
        drain_out(MAX_NC - 1, 1)

    return body


def _fin_common(i, feat_ref, mh_ref, w1a_ref, w1b_ref, gc_ref, starts_ref,
                ends_ref, gamma_ref, beta_ref, out_ref):
    mh = jnp.transpose(mh_ref[0])
    acc = jnp.dot(feat_ref[...], w1a_ref[...], preferred_element_type=jnp.float32)
    acc = acc + jnp.dot(mh, w1b_ref[...], preferred_element_type=jnp.float32)
    rows = lax.broadcasted_iota(jnp.int32, (BE, G), 0) + i * BE
    onehot = ((rows >= starts_ref[...]) & (rows < ends_ref[...])).astype(jnp.float32)
    acc = acc + jnp.dot(onehot, gc_ref[...], preferred_element_type=jnp.float32)
    hh = jnp.maximum(acc, 0.0)
    mu = jnp.mean(hh, axis=1, keepdims=True)
    dd = hh - mu
    var = jnp.mean(dd * dd, axis=1, keepdims=True)
    out_ref[...] = dd * lax.rsqrt(var + LN_EPS) * gamma_ref[...] + beta_ref[...]


def _make_fin_body(blk0, aliased):
    if aliased:
        def body(feat_ref, mh_ref, w1a_ref, w1b_ref, gc_ref, starts_ref,
                 ends_ref, gamma_ref, beta_ref, prev_ref, out_ref):
            del prev_ref
            _fin_common(pl.program_id(0) + blk0, feat_ref, mh_ref, w1a_ref,
                        w1b_ref, gc_ref, starts_ref, ends_ref, gamma_ref,
                        beta_ref, out_ref)
    else:
        def body(feat_ref, mh_ref, w1a_ref, w1b_ref, gc_ref, starts_ref,
                 ends_ref, gamma_ref, beta_ref, out_ref):
            _fin_common(pl.program_id(0) + blk0, feat_ref, mh_ref, w1a_ref,
                        w1b_ref, gc_ref, starts_ref, ends_ref, gamma_ref,
                        beta_ref, out_ref)
    return body


def kernel(feat, nodes, globals_, ind, num, Wq, bq, Wk, bk, conv_w, conv_b,
           W1, b1, ln_gamma, ln_beta):
    cw1 = conv_w[1]
    bq2 = bq.reshape(1, -1)
    bk2 = bk.reshape(1, -1)
    cb2 = conv_b.reshape(1, -1)
    w1a = W1[:D]
    w1b = W1[D:D + H]
    w1c = W1[D + H:]
    b12 = b1.reshape(1, -1)
    csum = jnp.cumsum(num.astype(jnp.int32))
    starts = (csum - num.astype(jnp.int32)).reshape(1, G)
    ends = csum.reshape(1, G)
    gamma2 = ln_gamma.reshape(1, -1)
    beta2 = ln_beta.reshape(1, -1)

    qmT = pl.pallas_call(
        _qm_body,
        grid=(NBE,),
        in_specs=[
            pl.BlockSpec((BE, D), lambda i: (i, 0)),
            pl.BlockSpec((D, H), lambda i: (0, 0)),
            pl.BlockSpec((H, H), lambda i: (0, 0)),
            pl.BlockSpec((1, H), lambda i: (0, 0)),
            pl.BlockSpec((1, H), lambda i: (0, 0)),
        ],
        out_specs=pl.BlockSpec((1, H, BE), lambda i: (i, 0, 0)),
        out_shape=jax.ShapeDtypeStruct((NBE, H, BE), jnp.float32),
    )(feat, Wq, cw1, bq2, cb2)

    kmT, gc = pl.pallas_call(
        _km_body,
        grid=(NBN,),
        in_specs=[
            pl.BlockSpec((BN, D), lambda i: (i, 0)),
            pl.BlockSpec((D, H), lambda i: (0, 0)),
            pl.BlockSpec((H, H), lambda i: (0, 0)),
            pl.BlockSpec((1, H), lambda i: (0, 0)),
            pl.BlockSpec((1, H), lambda i: (0, 0)),
            pl.BlockSpec((G, D), lambda i: (0, 0)),
            pl.BlockSpec((D, HID), lambda i: (0, 0)),
            pl.BlockSpec((1, HID), lambda i: (0, 0)),
        ],
        out_specs=[
            pl.BlockSpec((1, H, BN), lambda i: (i, 0, 0)),
            pl.BlockSpec((G, HID), lambda i: (0, 0)),
        ],
        out_shape=[
            jax.ShapeDtypeStruct((NBN, H, BN), jnp.float32),
            jax.ShapeDtypeStruct((G, HID), jnp.float32),
        ],
    )(nodes, Wk, cw1, bk2, cb2, globals_, w1c, b12)

    ind_flat = (
        ind.astype(jnp.int32)
        .reshape(NUM_CHUNKS, CHUNK, K)
        .transpose(0, 2, 1)
        .reshape(-1)
    )
    qmT_flat = qmT.reshape(-1)
    kmT_flat = kmT.reshape(-1)

    mh_splits = []
    for s in range(SPLITS):
        sc_attn = functools.partial(
            pl.kernel,
            mesh=plsc.VectorSubcoreMesh(core_axis_name="c", subcore_axis_name="s"),
            compiler_params=pltpu.CompilerParams(needs_layout_passes=False),
            out_type=jax.ShapeDtypeStruct((SPLIT_E * H,), jnp.float32),
            scratch_types=(
                [pltpu.VMEM((N,), jnp.float32)]
                + [pltpu.VMEM((CHUNK * K,), jnp.int32)] * 2
                + [pltpu.VMEM((CHUNK,), jnp.float32)] * 4
                + [pltpu.SemaphoreType.DMA] * 6
            ),
        )(_make_sc_body(s))
        mh_splits.append(
            sc_attn(ind_flat, qmT_flat, kmT_flat).reshape(SPB, H, BE)
        )

    base_specs = [
        pl.BlockSpec((D, HID), lambda i: (0, 0)),
        pl.BlockSpec((H, HID), lambda i: (0, 0)),
        pl.BlockSpec((G, HID), lambda i: (0, 0)),
        pl.BlockSpec((1, G), lambda i: (0, 0)),
        pl.BlockSpec((1, G), lambda i: (0, 0)),
        pl.BlockSpec((1, HID), lambda i: (0, 0)),
        pl.BlockSpec((1, HID), lambda i: (0, 0)),
    ]
    out = None
    for s in range(SPLITS):
        feat_spec = pl.BlockSpec((BE, D), lambda i, s=s: (i + s * SPB, 0))
        mh_spec = pl.BlockSpec((1, H, BE), lambda i: (i, 0, 0))
        out_spec = pl.BlockSpec((BE, HID), lambda i, s=s: (i + s * SPB, 0))
        args = [feat, mh_splits[s], w1a, w1b, gc, starts, ends, gamma2, beta2]
        in_specs = [feat_spec, mh_spec] + base_specs
        kwargs = {}
        if s > 0:
            args.append(out)
            in_specs.append(pl.BlockSpec(memory_space=pltpu.MemorySpace.HBM))
            kwargs["input_output_aliases"] = {9: 0}
        out = pl.pallas_call(
            _make_fin_body(s * SPB, aliased=s > 0),
            grid=(SPB,),
            in_specs=in_specs,
            out_specs=out_spec,
            out_shape=jax.ShapeDtypeStruct((E, HID), jnp.float32),
            **kwargs,
        )(*args)
    return out


# speedup vs baseline: 59.7709x; 1.3091x over previous
import functools
import math

import jax
import jax.numpy as jnp
from jax import lax
from jax.experimental import pallas as pl
from jax.experimental.pallas import tpu as pltpu
from jax.experimental.pallas import tpu_sc as plsc

E = 400000
N = 100000
G = 64
K = 16
D = 128
H = 8
HID = 128
LN_EPS = 1e-3

BE = 3200
NBE = E // BE
BN = 4000
NBN = N // BN
CHUNK = 640
CPB = BE // CHUNK
NUM_CHUNKS = E // CHUNK
NQ = 4
GROUPS = CHUNK // 16
SPLITS = 5
SPB = NBE // SPLITS
SPLIT_CHUNKS = NUM_CHUNKS // SPLITS
SPLIT_E = E // SPLITS
MAX_NC = (SPLIT_CHUNKS + NQ - 1) // NQ


def _qm_body(feat_ref, wq_ref, cw1_ref, bq_ref, cb_ref, out_ref):
    wqp = jnp.dot(wq_ref[...], cw1_ref[...], preferred_element_type=jnp.float32)
    bias = jnp.dot(bq_ref[...], cw1_ref[...], preferred_element_type=jnp.float32) + cb_ref[...]
    q = jnp.dot(feat_ref[...], wqp, preferred_element_type=jnp.float32) + bias
    out_ref[...] = jnp.transpose(q)[None, :, :]


def _km_body(nodes_ref, wk_ref, cw1_ref, bk_ref, cb_ref, glob_ref, w1c_ref, b1_ref,
             km_ref, gc_ref):
    i = pl.program_id(0)
    wkp = jnp.dot(wk_ref[...], cw1_ref[...], preferred_element_type=jnp.float32)
    bias = jnp.dot(bk_ref[...], cw1_ref[...], preferred_element_type=jnp.float32) + cb_ref[...]
    kmb = jnp.dot(nodes_ref[...], wkp, preferred_element_type=jnp.float32) + bias
    km_ref[...] = jnp.transpose(kmb)[None, :, :]

    @pl.when(i == 0)
    def _():
        gc_ref[...] = (
            jnp.dot(glob_ref[...], w1c_ref[...], preferred_element_type=jnp.float32)
            + b1_ref[...]
        )


def _make_sc_body(s):
    def body(ind_flat, qmT, kmT, out, tab_v, idx_v0, idx_v1, qh_v0, qh_v1,
             oh_v0, oh_v1, sem_i0, sem_i1, sem_q0, sem_q1, sem_o0, sem_o1,
             sem_t):
        idx_b = [idx_v0, idx_v1]
        qh_b = [qh_v0, qh_v1]
        oh_b = [oh_v0, oh_v1]
        sem_i = [sem_i0, sem_i1]
        sem_q = [sem_q0, sem_q1]
        sem_o = [sem_o0, sem_o1]
        wid = lax.axis_index("c") * 16 + lax.axis_index("s")
        h = wid % H
        qtr = wid // H

        cps = []
        for nb in range(NBN):
            cps.append(pltpu.async_copy(
                kmT.at[pl.ds((nb * H + h) * BN, BN)],
                tab_v.at[pl.ds(nb * BN, BN)], sem_t,
            ))
        for cp in cps:
            cp.wait()

        qscale = 1.0 / math.sqrt(float(H))
        n_c = (SPLIT_CHUNKS - qtr + NQ - 1) // NQ

        def srcs(j):
            lc = qtr + jnp.minimum(j, n_c - 1) * NQ
            c = s * SPLIT_CHUNKS + lc
            off = (lc % CPB) * CHUNK
            obase = ((lc // CPB) * H + h) * BE + off
            qbase = obase
            return c, qbase, obase

        def issue(j, b):
            c, qbase, _ = srcs(j)
            pltpu.async_copy(
                ind_flat.at[pl.ds(c * (CHUNK * K), CHUNK * K)], idx_b[b], sem_i[b]
            )
            pltpu.async_copy(qmT.at[pl.ds(qbase, CHUNK)], qh_b[b], sem_q[b])

        def drain(j, b):
            c, qbase, _ = srcs(j)
            pltpu.make_async_copy(
                ind_flat.at[pl.ds(c * (CHUNK * K), CHUNK * K)], idx_b[b], sem_i[b]
            ).wait()
            pltpu.make_async_copy(
                qmT.at[pl.ds(qbase, CHUNK)], qh_b[b], sem_q[b]
            ).wait()

        def compute(j, b):
            _, _, obase = srcs(j)

            def group_body(g, carry2):
                q2 = qh_b[b][pl.ds(g * 16, 16)] * qscale
                den = jnp.zeros((16,), jnp.float32)
                num = jnp.zeros((16,), jnp.float32)
                for k in range(K):
                    iv = idx_b[b][pl.ds(k * CHUNK + g * 16, 16)]
                    gk = plsc.load_gather(tab_v, [iv])
                    t = jnp.exp(q2 * gk)
                    den = den + t
                    num = num + t * gk
                oh_b[b][pl.ds(g * 16, 16)] = num / den
                return carry2

            lax.fori_loop(0, GROUPS, group_body, 0)
            pltpu.async_copy(oh_b[b], out.at[pl.ds(obase, CHUNK)], sem_o[b])

        def drain_out(j, b):
            _, _, obase = srcs(j)
            pltpu.make_async_copy(
                oh_b[b], out.at[pl.ds(obase, CHUNK)], sem_o[b]
            ).wait()

        issue(0, 0)

        def outer(i, carry):
            for b in range(2):
                j = i * 2 + b
                drain(j, b)
                issue(j + 1, 1 - b)

                @pl.when(j >= 2)
                def _():
                    drain_out(j - 2, b)

                compute(j, b)
            return carry

        lax.fori_loop(0, MAX_NC // 2, outer, 0)
        drain(MAX_NC, 0)
        drain_out(MAX_NC - 2, 0)
        drain_out(MAX_NC - 1, 1)

    return body


def _fin_common(i, feat_ref, mh_ref, w1a_ref, w1b_ref, gc_ref, starts_ref,
                ends_ref, gamma_ref, beta_ref, out_ref):
    mh = jnp.transpose(mh_ref[0])
    acc = jnp.dot(feat_ref[...], w1a_ref[...], preferred_element_type=jnp.float32)
    acc = acc + jnp.dot(mh, w1b_ref[...], preferred_element_type=jnp.float32)
    rows = lax.broadcasted_iota(jnp.int32, (BE, G), 0) + i * BE
    onehot = ((rows >= starts_ref[...]) & (rows < ends_ref[...])).astype(jnp.float32)
    acc = acc + jnp.dot(onehot, gc_ref[...], preferred_element_type=jnp.float32)
    hh = jnp.maximum(acc, 0.0)
    mu = jnp.mean(hh, axis=1, keepdims=True)
    dd = hh - mu
    var = jnp.mean(dd * dd, axis=1, keepdims=True)
    out_ref[...] = dd * lax.rsqrt(var + LN_EPS) * gamma_ref[...] + beta_ref[...]


def _make_fin_body(blk0, aliased):
    if aliased:
        def body(feat_ref, mh_ref, w1a_ref, w1b_ref, gc_ref, starts_ref,
                 ends_ref, gamma_ref, beta_ref, prev_ref, out_ref):
            del prev_ref
            _fin_common(pl.program_id(0) + blk0, feat_ref, mh_ref, w1a_ref,
                        w1b_ref, gc_ref, starts_ref, ends_ref, gamma_ref,
                        beta_ref, out_ref)
    else:
        def body(feat_ref, mh_ref, w1a_ref, w1b_ref, gc_ref, starts_ref,
                 ends_ref, gamma_ref, beta_ref, out_ref):
            _fin_common(pl.program_id(0) + blk0, feat_ref, mh_ref, w1a_ref,
                        w1b_ref, gc_ref, starts_ref, ends_ref, gamma_ref,
                        beta_ref, out_ref)
    return body


def kernel(feat, nodes, globals_, ind, num, Wq, bq, Wk, bk, conv_w, conv_b,
           W1, b1, ln_gamma, ln_beta):
    cw1 = conv_w[1]
    bq2 = bq.reshape(1, -1)
    bk2 = bk.reshape(1, -1)
    cb2 = conv_b.reshape(1, -1)
    w1a = W1[:D]
    w1b = W1[D:D + H]
    w1c = W1[D + H:]
    b12 = b1.reshape(1, -1)
    csum = jnp.cumsum(num.astype(jnp.int32))
    starts = (csum - num.astype(jnp.int32)).reshape(1, G)
    ends = csum.reshape(1, G)
    gamma2 = ln_gamma.reshape(1, -1)
    beta2 = ln_beta.reshape(1, -1)

    qmT_splits = []
    for s in range(SPLITS):
        qmT_s = pl.pallas_call(
            _qm_body,
            grid=(SPB,),
            in_specs=[
                pl.BlockSpec((BE, D), lambda i, s=s: (i + s * SPB, 0)),
                pl.BlockSpec((D, H), lambda i: (0, 0)),
                pl.BlockSpec((H, H), lambda i: (0, 0)),
                pl.BlockSpec((1, H), lambda i: (0, 0)),
                pl.BlockSpec((1, H), lambda i: (0, 0)),
            ],
            out_specs=pl.BlockSpec((1, H, BE), lambda i: (i, 0, 0)),
            out_shape=jax.ShapeDtypeStruct((SPB, H, BE), jnp.float32),
        )(feat, Wq, cw1, bq2, cb2)
        qmT_splits.append(qmT_s.reshape(-1))

    kmT, gc = pl.pallas_call(
        _km_body,
        grid=(NBN,),
        in_specs=[
            pl.BlockSpec((BN, D), lambda i: (i, 0)),
            pl.BlockSpec((D, H), lambda i: (0, 0)),
            pl.BlockSpec((H, H), lambda i: (0, 0)),
            pl.BlockSpec((1, H), lambda i: (0, 0)),
            pl.BlockSpec((1, H), lambda i: (0, 0)),
            pl.BlockSpec((G, D), lambda i: (0, 0)),
            pl.BlockSpec((D, HID), lambda i: (0, 0)),
            pl.BlockSpec((1, HID), lambda i: (0, 0)),
        ],
        out_specs=[
            pl.BlockSpec((1, H, BN), lambda i: (i, 0, 0)),
            pl.BlockSpec((G, HID), lambda i: (0, 0)),
        ],
        out_shape=[
            jax.ShapeDtypeStruct((NBN, H, BN), jnp.float32),
            jax.ShapeDtypeStruct((G, HID), jnp.float32),
        ],
    )(nodes, Wk, cw1, bk2, cb2, globals_, w1c, b12)

    ind_flat = (
        ind.astype(jnp.int32)
        .reshape(NUM_CHUNKS, CHUNK, K)
        .transpose(0, 2, 1)
        .reshape(-1)
    )
    kmT_flat = kmT.reshape(-1)

    mh_splits = []
    for s in range(SPLITS):
        sc_attn = functools.partial(
            pl.kernel,
            mesh=plsc.VectorSubcoreMesh(core_axis_name="c", subcore_axis_name="s"),
            compiler_params=pltpu.CompilerParams(needs_layout_passes=False),
            out_type=jax.ShapeDtypeStruct((SPLIT_E * H,), jnp.float32),
            scratch_types=(
                [pltpu.VMEM((N,), jnp.float32)]
                + [pltpu.VMEM((CHUNK * K,), jnp.int32)] * 2
                + [pltpu.VMEM((CHUNK,), jnp.float32)] * 4
                + [pltpu.SemaphoreType.DMA] * 7
            ),
        )(_make_sc_body(s))
        mh_splits.append(
            sc_attn(ind_flat, qmT_splits[s], kmT_flat).reshape(SPB, H, BE)
        )

    base_specs = [
        pl.BlockSpec((D, HID), lambda i: (0, 0)),
        pl.BlockSpec((H, HID), lambda i: (0, 0)),
        pl.BlockSpec((G, HID), lambda i: (0, 0)),
        pl.BlockSpec((1, G), lambda i: (0, 0)),
        pl.BlockSpec((1, G), lambda i: (0, 0)),
        pl.BlockSpec((1, HID), lambda i: (0, 0)),
        pl.BlockSpec((1, HID), lambda i: (0, 0)),
    ]
    out = None
    for s in range(SPLITS):
        feat_spec = pl.BlockSpec((BE, D), lambda i, s=s: (i + s * SPB, 0))
        mh_spec = pl.BlockSpec((1, H, BE), lambda i: (i, 0, 0))
        out_spec = pl.BlockSpec((BE, HID), lambda i, s=s: (i + s * SPB, 0))
        args = [feat, mh_splits[s], w1a, w1b, gc, starts, ends, gamma2, beta2]
        in_specs = [feat_spec, mh_spec] + base_specs
        kwargs = {}
        if s > 0:
            args.append(out)
            in_specs.append(pl.BlockSpec(memory_space=pltpu.MemorySpace.HBM))
            kwargs["input_output_aliases"] = {9: 0}
        out = pl.pallas_call(
            _make_fin_body(s * SPB, aliased=s > 0),
            grid=(SPB,),
            in_specs=in_specs,
            out_specs=out_spec,
            out_shape=jax.ShapeDtypeStruct((E, HID), jnp.float32),
            **kwargs,
        )(*args)
    return out


# speedup vs baseline: 61.2551x; 1.0248x over previous
import functools
import math

import jax
import jax.numpy as jnp
from jax import lax
from jax.experimental import pallas as pl
from jax.experimental.pallas import tpu as pltpu
from jax.experimental.pallas import tpu_sc as plsc

E = 400000
N = 100000
G = 64
K = 16
D = 128
H = 8
HID = 128
LN_EPS = 1e-3

BE = 3200
NBE = E // BE
BN = 4000
NBN = N // BN
CHUNK = 640
CPB = BE // CHUNK
NUM_CHUNKS = E // CHUNK
NQ = 4
GROUPS = CHUNK // 16
SPLITS = 5
SPB = NBE // SPLITS
SPLIT_CHUNKS = NUM_CHUNKS // SPLITS
SPLIT_E = E // SPLITS
MAX_NC = (SPLIT_CHUNKS + NQ - 1) // NQ


def _qm_body(feat_ref, wq_ref, cw1_ref, bq_ref, cb_ref, out_ref):
    wqp = jnp.dot(wq_ref[...], cw1_ref[...], preferred_element_type=jnp.float32)
    bias = jnp.dot(bq_ref[...], cw1_ref[...], preferred_element_type=jnp.float32) + cb_ref[...]
    qt = lax.dot_general(
        wqp, feat_ref[...], (((0,), (1,)), ((), ())),
        preferred_element_type=jnp.float32,
    ) + jnp.transpose(bias)
    out_ref[...] = qt[None, :, :]


def _km_body(nodes_ref, wk_ref, cw1_ref, bk_ref, cb_ref, glob_ref, w1c_ref, b1_ref,
             km_ref, gc_ref):
    i = pl.program_id(0)
    wkp = jnp.dot(wk_ref[...], cw1_ref[...], preferred_element_type=jnp.float32)
    bias = jnp.dot(bk_ref[...], cw1_ref[...], preferred_element_type=jnp.float32) + cb_ref[...]
    kt = lax.dot_general(
        wkp, nodes_ref[...], (((0,), (1,)), ((), ())),
        preferred_element_type=jnp.float32,
    ) + jnp.transpose(bias)
    km_ref[...] = kt[None, :, :]

    @pl.when(i == 0)
    def _():
        gc_ref[...] = (
            jnp.dot(glob_ref[...], w1c_ref[...], preferred_element_type=jnp.float32)
            + b1_ref[...]
        )


def _make_sc_body(s):
    def body(ind_flat, qmT, kmT, out, tab_v, idx_v0, idx_v1, qh_v0, qh_v1,
             oh_v0, oh_v1, sem_i0, sem_i1, sem_q0, sem_q1, sem_o0, sem_o1,
             sem_t):
        idx_b = [idx_v0, idx_v1]
        qh_b = [qh_v0, qh_v1]
        oh_b = [oh_v0, oh_v1]
        sem_i = [sem_i0, sem_i1]
        sem_q = [sem_q0, sem_q1]
        sem_o = [sem_o0, sem_o1]
        wid = lax.axis_index("c") * 16 + lax.axis_index("s")
        h = wid % H
        qtr = wid // H

        cps = []
        for nb in range(NBN):
            cps.append(pltpu.async_copy(
                kmT.at[pl.ds((nb * H + h) * BN, BN)],
                tab_v.at[pl.ds(nb * BN, BN)], sem_t,
            ))
        for cp in cps:
            cp.wait()

        qscale = 1.0 / math.sqrt(float(H))
        n_c = (SPLIT_CHUNKS - qtr + NQ - 1) // NQ

        def srcs(j):
            lc = qtr + jnp.minimum(j, n_c - 1) * NQ
            c = s * SPLIT_CHUNKS + lc
            off = (lc % CPB) * CHUNK
            obase = ((lc // CPB) * H + h) * BE + off
            qbase = obase
            return c, qbase, obase

        def issue(j, b):
            c, qbase, _ = srcs(j)
            pltpu.async_copy(
                ind_flat.at[pl.ds(c * (CHUNK * K), CHUNK * K)], idx_b[b], sem_i[b]
            )
            pltpu.async_copy(qmT.at[pl.ds(qbase, CHUNK)], qh_b[b], sem_q[b])

        def drain(j, b):
            c, qbase, _ = srcs(j)
            pltpu.make_async_copy(
                ind_flat.at[pl.ds(c * (CHUNK * K), CHUNK * K)], idx_b[b], sem_i[b]
            ).wait()
            pltpu.make_async_copy(
                qmT.at[pl.ds(qbase, CHUNK)], qh_b[b], sem_q[b]
            ).wait()

        def compute(j, b):
            _, _, obase = srcs(j)

            def group_body(g, carry2):
                q2 = qh_b[b][pl.ds(g * 16, 16)] * qscale
                den = jnp.zeros((16,), jnp.float32)
                num = jnp.zeros((16,), jnp.float32)
                for k in range(K):
                    iv = idx_b[b][pl.ds(k * CHUNK + g * 16, 16)]
                    gk = plsc.load_gather(tab_v, [iv])
                    t = jnp.exp(q2 * gk)
                    den = den + t
                    num = num + t * gk
                oh_b[b][pl.ds(g * 16, 16)] = num / den
                return carry2

            lax.fori_loop(0, GROUPS, group_body, 0)
            pltpu.async_copy(oh_b[b], out.at[pl.ds(obase, CHUNK)], sem_o[b])

        def drain_out(j, b):
            _, _, obase = srcs(j)
            pltpu.make_async_copy(
                oh_b[b], out.at[pl.ds(obase, CHUNK)], sem_o[b]
            ).wait()

        issue(0, 0)

        def outer(i, carry):
            for b in range(2):
                j = i * 2 + b
                drain(j, b)
                issue(j + 1, 1 - b)

                @pl.when(j >= 2)
                def _():
                    drain_out(j - 2, b)

                compute(j, b)
            return carry

        lax.fori_loop(0, MAX_NC // 2, outer, 0)
        drain(MAX_NC, 0)
        drain_out(MAX_NC - 2, 0)
        drain_out(MAX_NC - 1, 1)

    return body


def _fin_common(i, feat_ref, mh_ref, w1a_ref, w1b_ref, gc_ref, starts_ref,
                ends_ref, gamma_ref, beta_ref, out_ref):
    mh = jnp.transpose(mh_ref[0])
    acc = jnp.dot(feat_ref[...], w1a_ref[...], preferred_element_type=jnp.float32)
    acc = acc + jnp.dot(mh, w1b_ref[...], preferred_element_type=jnp.float32)
    rows = lax.broadcasted_iota(jnp.int32, (BE, G), 0) + i * BE
    onehot = ((rows >= starts_ref[...]) & (rows < ends_ref[...])).astype(jnp.float32)
    acc = acc + jnp.dot(onehot, gc_ref[...], preferred_element_type=jnp.float32)
    hh = jnp.maximum(acc, 0.0)
    mu = jnp.mean(hh, axis=1, keepdims=True)
    dd = hh - mu
    var = jnp.mean(dd * dd, axis=1, keepdims=True)
    out_ref[...] = dd * lax.rsqrt(var + LN_EPS) * gamma_ref[...] + beta_ref[...]


def _make_fin_body(blk0, aliased):
    if aliased:
        def body(feat_ref, mh_ref, w1a_ref, w1b_ref, gc_ref, starts_ref,
                 ends_ref, gamma_ref, beta_ref, prev_ref, out_ref):
            del prev_ref
            _fin_common(pl.program_id(0) + blk0, feat_ref, mh_ref, w1a_ref,
                        w1b_ref, gc_ref, starts_ref, ends_ref, gamma_ref,
                        beta_ref, out_ref)
    else:
        def body(feat_ref, mh_ref, w1a_ref, w1b_ref, gc_ref, starts_ref,
                 ends_ref, gamma_ref, beta_ref, out_ref):
            _fin_common(pl.program_id(0) + blk0, feat_ref, mh_ref, w1a_ref,
                        w1b_ref, gc_ref, starts_ref, ends_ref, gamma_ref,
                        beta_ref, out_ref)
    return body


def kernel(feat, nodes, globals_, ind, num, Wq, bq, Wk, bk, conv_w, conv_b,
           W1, b1, ln_gamma, ln_beta):
    cw1 = conv_w[1]
    bq2 = bq.reshape(1, -1)
    bk2 = bk.reshape(1, -1)
    cb2 = conv_b.reshape(1, -1)
    w1a = W1[:D]
    w1b = W1[D:D + H]
    w1c = W1[D + H:]
    b12 = b1.reshape(1, -1)
    csum = jnp.cumsum(num.astype(jnp.int32))
    starts = (csum - num.astype(jnp.int32)).reshape(1, G)
    ends = csum.reshape(1, G)
    gamma2 = ln_gamma.reshape(1, -1)
    beta2 = ln_beta.reshape(1, -1)

    qmT_splits = []
    for s in range(SPLITS):
        qmT_s = pl.pallas_call(
            _qm_body,
            grid=(SPB,),
            in_specs=[
                pl.BlockSpec((BE, D), lambda i, s=s: (i + s * SPB, 0)),
                pl.BlockSpec((D, H), lambda i: (0, 0)),
                pl.BlockSpec((H, H), lambda i: (0, 0)),
                pl.BlockSpec((1, H), lambda i: (0, 0)),
                pl.BlockSpec((1, H), lambda i: (0, 0)),
            ],
            out_specs=pl.BlockSpec((1, H, BE), lambda i: (i, 0, 0)),
            out_shape=jax.ShapeDtypeStruct((SPB, H, BE), jnp.float32),
        )(feat, Wq, cw1, bq2, cb2)
        qmT_splits.append(qmT_s.reshape(-1))

    kmT, gc = pl.pallas_call(
        _km_body,
        grid=(NBN,),
        in_specs=[
            pl.BlockSpec((BN, D), lambda i: (i, 0)),
            pl.BlockSpec((D, H), lambda i: (0, 0)),
            pl.BlockSpec((H, H), lambda i: (0, 0)),
            pl.BlockSpec((1, H), lambda i: (0, 0)),
            pl.BlockSpec((1, H), lambda i: (0, 0)),
            pl.BlockSpec((G, D), lambda i: (0, 0)),
            pl.BlockSpec((D, HID), lambda i: (0, 0)),
            pl.BlockSpec((1, HID), lambda i: (0, 0)),
        ],
        out_specs=[
            pl.BlockSpec((1, H, BN), lambda i: (i, 0, 0)),
            pl.BlockSpec((G, HID), lambda i: (0, 0)),
        ],
        out_shape=[
            jax.ShapeDtypeStruct((NBN, H, BN), jnp.float32),
            jax.ShapeDtypeStruct((G, HID), jnp.float32),
        ],
    )(nodes, Wk, cw1, bk2, cb2, globals_, w1c, b12)

    ind_flat = (
        ind.astype(jnp.int32)
        .reshape(NUM_CHUNKS, CHUNK, K)
        .transpose(0, 2, 1)
        .reshape(-1)
    )
    kmT_flat = kmT.reshape(-1)

    mh_splits = []
    for s in range(SPLITS):
        sc_attn = functools.partial(
            pl.kernel,
            mesh=plsc.VectorSubcoreMesh(core_axis_name="c", subcore_axis_name="s"),
            compiler_params=pltpu.CompilerParams(needs_layout_passes=False),
            out_type=jax.ShapeDtypeStruct((SPLIT_E * H,), jnp.float32),
            scratch_types=(
                [pltpu.VMEM((N,), jnp.float32)]
                + [pltpu.VMEM((CHUNK * K,), jnp.int32)] * 2
                + [pltpu.VMEM((CHUNK,), jnp.float32)] * 4
                + [pltpu.SemaphoreType.DMA] * 7
            ),
        )(_make_sc_body(s))
        mh_splits.append(
            sc_attn(ind_flat, qmT_splits[s], kmT_flat).reshape(SPB, H, BE)
        )

    base_specs = [
        pl.BlockSpec((D, HID), lambda i: (0, 0)),
        pl.BlockSpec((H, HID), lambda i: (0, 0)),
        pl.BlockSpec((G, HID), lambda i: (0, 0)),
        pl.BlockSpec((1, G), lambda i: (0, 0)),
        pl.BlockSpec((1, G), lambda i: (0, 0)),
        pl.BlockSpec((1, HID), lambda i: (0, 0)),
        pl.BlockSpec((1, HID), lambda i: (0, 0)),
    ]
    out = None
    for s in range(SPLITS):
        feat_spec = pl.BlockSpec((BE, D), lambda i, s=s: (i + s * SPB, 0))
        mh_spec = pl.BlockSpec((1, H, BE), lambda i: (i, 0, 0))
        out_spec = pl.BlockSpec((BE, HID), lambda i, s=s: (i + s * SPB, 0))
        args = [feat, mh_splits[s], w1a, w1b, gc, starts, ends, gamma2, beta2]
        in_specs = [feat_spec, mh_spec] + base_specs
        kwargs = {}
        if s > 0:
            args.append(out)
            in_specs.append(pl.BlockSpec(memory_space=pltpu.MemorySpace.HBM))
            kwargs["input_output_aliases"] = {9: 0}
        out = pl.pallas_call(
            _make_fin_body(s * SPB, aliased=s > 0),
            grid=(SPB,),
            in_specs=in_specs,
            out_specs=out_spec,
            out_shape=jax.ShapeDtypeStruct((E, HID), jnp.float32),
            **kwargs,
        )(*args)
    return out


# speedup vs baseline: 61.2676x; 1.0002x over previous
import functools
import math

import jax
import jax.numpy as jnp
from jax import lax
from jax.experimental import pallas as pl
from jax.experimental.pallas import tpu as pltpu
from jax.experimental.pallas import tpu_sc as plsc

E = 400000
N = 100000
G = 64
K = 16
D = 128
H = 8
HID = 128
LN_EPS = 1e-3

BE = 3200
NBE = E // BE
BN = 4000
NBN = N // BN
CHUNK = 640
CPB = BE // CHUNK
NUM_CHUNKS = E // CHUNK
NQ = 4
GROUPS = CHUNK // 16
SPLITS = 5
SPB = NBE // SPLITS
SPLIT_CHUNKS = NUM_CHUNKS // SPLITS
SPLIT_E = E // SPLITS
MAX_NC = (SPLIT_CHUNKS + NQ - 1) // NQ


def _qm_body(feat_ref, wq_ref, cw1_ref, bq_ref, cb_ref, out_ref):
    wqp = jnp.dot(wq_ref[...], cw1_ref[...], preferred_element_type=jnp.float32)
    bias = jnp.dot(bq_ref[...], cw1_ref[...], preferred_element_type=jnp.float32) + cb_ref[...]
    qt = lax.dot_general(
        wqp, feat_ref[...], (((0,), (1,)), ((), ())),
        preferred_element_type=jnp.float32,
    ) + jnp.transpose(bias)
    out_ref[...] = qt[None, :, :]


def _km_body(nodes_ref, wk_ref, cw1_ref, bk_ref, cb_ref, glob_ref, w1c_ref, b1_ref,
             km_ref, gc_ref):
    i = pl.program_id(0)
    wkp = jnp.dot(wk_ref[...], cw1_ref[...], preferred_element_type=jnp.float32)
    bias = jnp.dot(bk_ref[...], cw1_ref[...], preferred_element_type=jnp.float32) + cb_ref[...]
    kt = lax.dot_general(
        wkp, nodes_ref[...], (((0,), (1,)), ((), ())),
        preferred_element_type=jnp.float32,
    ) + jnp.transpose(bias)
    km_ref[...] = kt[None, :, :]

    @pl.when(i == 0)
    def _():
        gc_ref[...] = (
            jnp.dot(glob_ref[...], w1c_ref[...], preferred_element_type=jnp.float32)
            + b1_ref[...]
        )


def _make_sc_body(s):
    def body(ind_flat, qmT, kmT, out, tab_v, idx_v0, idx_v1, qh_v0, qh_v1,
             oh_v0, oh_v1, sem_i0, sem_i1, sem_q0, sem_q1, sem_o0, sem_o1,
             sem_t):
        idx_b = [idx_v0, idx_v1]
        qh_b = [qh_v0, qh_v1]
        oh_b = [oh_v0, oh_v1]
        sem_i = [sem_i0, sem_i1]
        sem_q = [sem_q0, sem_q1]
        sem_o = [sem_o0, sem_o1]
        wid = lax.axis_index("c") * 16 + lax.axis_index("s")
        h = wid % H
        qtr = wid // H

        cps = []
        for nb in range(NBN):
            cps.append(pltpu.async_copy(
                kmT.at[pl.ds((nb * H + h) * BN, BN)],
                tab_v.at[pl.ds(nb * BN, BN)], sem_t,
            ))
        for cp in cps:
            cp.wait()

        qscale = 1.0 / math.sqrt(float(H))
        n_c = (SPLIT_CHUNKS - qtr + NQ - 1) // NQ

        def srcs(j):
            lc = qtr + jnp.minimum(j, n_c - 1) * NQ
            c = s * SPLIT_CHUNKS + lc
            nb_l = lc // CPB
            off = (lc % CPB) * CHUNK
            return c, nb_l, off

        def issue(j, b):
            c, nb_l, off = srcs(j)
            pltpu.async_copy(
                ind_flat.at[pl.ds(c * (CHUNK * K), CHUNK * K)], idx_b[b], sem_i[b]
            )
            for t in range(CHUNK // 128):
                pltpu.async_copy(
                    qmT.at[nb_l, h, pl.ds(off + t * 128, 128)],
                    qh_b[b].at[pl.ds(t * 128, 128)], sem_q[b],
                )

        def drain(j, b):
            c, nb_l, off = srcs(j)
            pltpu.make_async_copy(
                ind_flat.at[pl.ds(c * (CHUNK * K), CHUNK * K)], idx_b[b], sem_i[b]
            ).wait()
            for t in range(CHUNK // 128):
                pltpu.make_async_copy(
                    qmT.at[nb_l, h, pl.ds(off + t * 128, 128)],
                    qh_b[b].at[pl.ds(t * 128, 128)], sem_q[b],
                ).wait()

        def compute(j, b):
            _, nb_l, off = srcs(j)

            def group_body(g, carry2):
                q2 = qh_b[b][pl.ds(g * 16, 16)] * qscale
                den = jnp.zeros((16,), jnp.float32)
                num = jnp.zeros((16,), jnp.float32)
                for k in range(K):
                    iv = idx_b[b][pl.ds(k * CHUNK + g * 16, 16)]
                    gk = plsc.load_gather(tab_v, [iv])
                    t = jnp.exp(q2 * gk)
                    den = den + t
                    num = num + t * gk
                oh_b[b][pl.ds(g * 16, 16)] = num / den
                return carry2

            lax.fori_loop(0, GROUPS, group_body, 0)
            for t in range(CHUNK // 128):
                pltpu.async_copy(
                    oh_b[b].at[pl.ds(t * 128, 128)],
                    out.at[nb_l, h, pl.ds(off + t * 128, 128)], sem_o[b],
                )

        def drain_out(j, b):
            _, nb_l, off = srcs(j)
            for t in range(CHUNK // 128):
                pltpu.make_async_copy(
                    oh_b[b].at[pl.ds(t * 128, 128)],
                    out.at[nb_l, h, pl.ds(off + t * 128, 128)], sem_o[b],
                ).wait()

        issue(0, 0)

        def outer(i, carry):
            for b in range(2):
                j = i * 2 + b
                drain(j, b)
                issue(j + 1, 1 - b)

                @pl.when(j >= 2)
                def _():
                    drain_out(j - 2, b)

                compute(j, b)
            return carry

        lax.fori_loop(0, MAX_NC // 2, outer, 0)
        drain(MAX_NC, 0)
        drain_out(MAX_NC - 2, 0)
        drain_out(MAX_NC - 1, 1)

    return body


def _fin_common(i, feat_ref, mh_ref, w1a_ref, w1b_ref, gc_ref, starts_ref,
                ends_ref, gamma_ref, beta_ref, out_ref):
    mh = jnp.transpose(mh_ref[0])
    acc = jnp.dot(feat_ref[...], w1a_ref[...], preferred_element_type=jnp.float32)
    acc = acc + jnp.dot(mh, w1b_ref[...], preferred_element_type=jnp.float32)
    rows = lax.broadcasted_iota(jnp.int32, (BE, G), 0) + i * BE
    onehot = ((rows >= starts_ref[...]) & (rows < ends_ref[...])).astype(jnp.float32)
    acc = acc + jnp.dot(onehot, gc_ref[...], preferred_element_type=jnp.float32)
    hh = jnp.maximum(acc, 0.0)
    mu = jnp.mean(hh, axis=1, keepdims=True)
    dd = hh - mu
    var = jnp.mean(dd * dd, axis=1, keepdims=True)
    out_ref[...] = dd * lax.rsqrt(var + LN_EPS) * gamma_ref[...] + beta_ref[...]


def _make_fin_body(blk0, aliased):
    if aliased:
        def body(feat_ref, mh_ref, w1a_ref, w1b_ref, gc_ref, starts_ref,
                 ends_ref, gamma_ref, beta_ref, prev_ref, out_ref):
            del prev_ref
            _fin_common(pl.program_id(0) + blk0, feat_ref, mh_ref, w1a_ref,
                        w1b_ref, gc_ref, starts_ref, ends_ref, gamma_ref,
                        beta_ref, out_ref)
    else:
        def body(feat_ref, mh_ref, w1a_ref, w1b_ref, gc_ref, starts_ref,
                 ends_ref, gamma_ref, beta_ref, out_ref):
            _fin_common(pl.program_id(0) + blk0, feat_ref, mh_ref, w1a_ref,
                        w1b_ref, gc_ref, starts_ref, ends_ref, gamma_ref,
                        beta_ref, out_ref)
    return body


def kernel(feat, nodes, globals_, ind, num, Wq, bq, Wk, bk, conv_w, conv_b,
           W1, b1, ln_gamma, ln_beta):
    cw1 = conv_w[1]
    bq2 = bq.reshape(1, -1)
    bk2 = bk.reshape(1, -1)
    cb2 = conv_b.reshape(1, -1)
    w1a = W1[:D]
    w1b = W1[D:D + H]
    w1c = W1[D + H:]
    b12 = b1.reshape(1, -1)
    csum = jnp.cumsum(num.astype(jnp.int32))
    starts = (csum - num.astype(jnp.int32)).reshape(1, G)
    ends = csum.reshape(1, G)
    gamma2 = ln_gamma.reshape(1, -1)
    beta2 = ln_beta.reshape(1, -1)

    qmT_splits = []
    for s in range(SPLITS):
        qmT_s = pl.pallas_call(
            _qm_body,
            grid=(SPB,),
            in_specs=[
                pl.BlockSpec((BE, D), lambda i, s=s: (i + s * SPB, 0)),
                pl.BlockSpec((D, H), lambda i: (0, 0)),
                pl.BlockSpec((H, H), lambda i: (0, 0)),
                pl.BlockSpec((1, H), lambda i: (0, 0)),
                pl.BlockSpec((1, H), lambda i: (0, 0)),
            ],
            out_specs=pl.BlockSpec((1, H, BE), lambda i: (i, 0, 0)),
            out_shape=jax.ShapeDtypeStruct((SPB, H, BE), jnp.float32),
        )(feat, Wq, cw1, bq2, cb2)
        qmT_splits.append(qmT_s)

    kmT, gc = pl.pallas_call(
        _km_body,
        grid=(NBN,),
        in_specs=[
            pl.BlockSpec((BN, D), lambda i: (i, 0)),
            pl.BlockSpec((D, H), lambda i: (0, 0)),
            pl.BlockSpec((H, H), lambda i: (0, 0)),
            pl.BlockSpec((1, H), lambda i: (0, 0)),
            pl.BlockSpec((1, H), lambda i: (0, 0)),
            pl.BlockSpec((G, D), lambda i: (0, 0)),
            pl.BlockSpec((D, HID), lambda i: (0, 0)),
            pl.BlockSpec((1, HID), lambda i: (0, 0)),
        ],
        out_specs=[
            pl.BlockSpec((1, H, BN), lambda i: (i, 0, 0)),
            pl.BlockSpec((G, HID), lambda i: (0, 0)),
        ],
        out_shape=[
            jax.ShapeDtypeStruct((NBN, H, BN), jnp.float32),
            jax.ShapeDtypeStruct((G, HID), jnp.float32),
        ],
    )(nodes, Wk, cw1, bk2, cb2, globals_, w1c, b12)

    ind_flat = (
        ind.astype(jnp.int32)
        .reshape(NUM_CHUNKS, CHUNK, K)
        .transpose(0, 2, 1)
        .reshape(-1)
    )
    kmT_flat = kmT.reshape(-1)

    mh_splits = []
    for s in range(SPLITS):
        sc_attn = functools.partial(
            pl.kernel,
            mesh=plsc.VectorSubcoreMesh(core_axis_name="c", subcore_axis_name="s"),
            compiler_params=pltpu.CompilerParams(needs_layout_passes=False),
            out_type=jax.ShapeDtypeStruct((SPB, H, BE), jnp.float32),
            scratch_types=(
                [pltpu.VMEM((N,), jnp.float32)]
                + [pltpu.VMEM((CHUNK * K,), jnp.int32)] * 2
                + [pltpu.VMEM((CHUNK,), jnp.float32)] * 4
                + [pltpu.SemaphoreType.DMA] * 7
            ),
        )(_make_sc_body(s))
        mh_splits.append(sc_attn(ind_flat, qmT_splits[s], kmT_flat))

    base_specs = [
        pl.BlockSpec((D, HID), lambda i: (0, 0)),
        pl.BlockSpec((H, HID), lambda i: (0, 0)),
        pl.BlockSpec((G, HID), lambda i: (0, 0)),
        pl.BlockSpec((1, G), lambda i: (0, 0)),
        pl.BlockSpec((1, G), lambda i: (0, 0)),
        pl.BlockSpec((1, HID), lambda i: (0, 0)),
        pl.BlockSpec((1, HID), lambda i: (0, 0)),
    ]
    out = None
    for s in range(SPLITS):
        feat_spec = pl.BlockSpec((BE, D), lambda i, s=s: (i + s * SPB, 0))
        mh_spec = pl.BlockSpec((1, H, BE), lambda i: (i, 0, 0))
        out_spec = pl.BlockSpec((BE, HID), lambda i, s=s: (i + s * SPB, 0))
        args = [feat, mh_splits[s], w1a, w1b, gc, starts, ends, gamma2, beta2]
        in_specs = [feat_spec, mh_spec] + base_specs
        kwargs = {}
        if s > 0:
            args.append(out)
            in_specs.append(pl.BlockSpec(memory_space=pltpu.MemorySpace.HBM))
            kwargs["input_output_aliases"] = {9: 0}
        out = pl.pallas_call(
            _make_fin_body(s * SPB, aliased=s > 0),
            grid=(SPB,),
            in_specs=in_specs,
            out_specs=out_spec,
            out_shape=jax.ShapeDtypeStruct((E, HID), jnp.float32),
            **kwargs,
        )(*args)
    return out
